# baseline (speedup 1.0000x reference)
import functools

import jax
import jax.numpy as jnp
from jax import lax
from jax.experimental import pallas as pl
from jax.experimental.pallas import tpu as pltpu
from jax.experimental.pallas import tpu_sc as plsc

N = 50000
K = 256
Q_MIN = 0.1
S_B = 1.0
E_DEN_OFF = 1.0
BLK = 12800
NB = 4
N_PAD = NB * BLK
BIG = 2 ** 30
NW = 16
CH = N_PAD // NW
NV = CH // 16


def _sc_body(tidx_hbm, beta_hbm, x0_hbm, x1_hbm,
             segmax_o, pwsum_o, aidx_o, xa0_o, xa1_o, ncnt_o, nbeta_o,
             winv_o,
             tidx_v, beta_v, x0_v, x1_v, winv_v,
             binm, binp, bina, binx0, binx1,
             segfull, pwfull, locf, loci, locx0, locx1,
             comb, combi, combx0, combx1, nc_v, nb_v, combn,
             shm_max, shm_pw, shm_amin, shm_x0, shm_x1, shm_nc, shm_nb):
    wid = lax.axis_index("s")
    base = wid * CH
    lane16 = lax.broadcasted_iota(jnp.int32, (16,), 0)

    pltpu.sync_copy(tidx_hbm.at[pl.ds(base, CH)], tidx_v)
    pltpu.sync_copy(beta_hbm.at[pl.ds(base, CH)], beta_v)
    pltpu.sync_copy(x0_hbm.at[pl.ds(base, CH)], x0_v)
    pltpu.sync_copy(x1_hbm.at[pl.ds(base, CH)], x1_v)

    def init_bins(i, _):
        sl = pl.ds(i * 16, 16)
        binm[sl] = jnp.full((16,), -1.0, jnp.float32)
        binp[sl] = jnp.zeros((16,), jnp.float32)
        bina[sl] = jnp.full((16,), BIG, jnp.int32)
        binx0[sl] = jnp.zeros((16,), jnp.float32)
        binx1[sl] = jnp.zeros((16,), jnp.float32)
        return 0
    lax.fori_loop(0, K, init_bins, 0)

    def p1(j, carry):
        nc, nb = carry
        sl = pl.ds(j * 16, 16)
        tv = tidx_v[sl]
        bv = jnp.minimum(jnp.maximum(beta_v[sl], 1e-6), 1.0 - 1e-4)
        noise = tv < 0
        obj = jnp.where(noise, 0, tv)
        idx16 = lane16 * K + obj
        val = jnp.where(noise, -1.0, bv)
        cur = plsc.load_gather(binm, [idx16])
        plsc.store_scatter(binm, [idx16], jnp.maximum(cur, val))
        pw = jnp.where(noise, 0.0, bv * bv)
        curp = plsc.load_gather(binp, [idx16])
        plsc.store_scatter(binp, [idx16], curp + pw)
        gi = base + j * 16 + lane16
        nf = jnp.where(jnp.logical_and(noise, gi < N), 1.0, 0.0)
        return (nc + nf, nb + nf * bv)

    nc, nb = lax.fori_loop(0, NV, p1, (jnp.zeros((16,), jnp.float32),
                                       jnp.zeros((16,), jnp.float32)))

    for kb in range(16):
        sl = pl.ds(kb * 16, 16)
        am = binm[sl]
        ap = binp[sl]
        for c in range(1, 16):
            sc = pl.ds(c * K + kb * 16, 16)
            am = jnp.maximum(am, binm[sc])
            ap = ap + binp[sc]
        segfull[sl] = am
        binp[sl] = ap

    nc_v[...] = nc
    nb_v[...] = nb
    pltpu.sync_copy(segfull, shm_max.at[wid])
    pltpu.sync_copy(binp.at[pl.ds(0, K)], shm_pw.at[wid])
    pltpu.sync_copy(nc_v, shm_nc.at[wid])
    pltpu.sync_copy(nb_v, shm_nb.at[wid])
    plsc.subcore_barrier()

    pltpu.sync_copy(shm_max, comb)
    for kb in range(16):
        sl = pl.ds(kb * 16, 16)
        am = comb[0, sl]
        for r in range(1, NW):
            am = jnp.maximum(am, comb[r, sl])
        segfull[sl] = am
    pltpu.sync_copy(shm_pw, comb)
    for kb in range(16):
        sl = pl.ds(kb * 16, 16)
        ap = comb[0, sl]
        for r in range(1, NW):
            ap = ap + comb[r, sl]
        pwfull[sl] = ap

    def p2(j, _):
        sl = pl.ds(j * 16, 16)
        tv = tidx_v[sl]
        bv = jnp.minimum(jnp.maximum(beta_v[sl], 1e-6), 1.0 - 1e-4)
        noise = tv < 0
        obj = jnp.where(noise, 0, tv)
        idx16 = lane16 * K + obj
        smax = plsc.load_gather(segfull, [obj])
        gi = base + j * 16 + lane16
        isal = jnp.logical_and(jnp.logical_not(noise), bv >= smax)
        key = jnp.where(isal, gi, BIG)
        cur = plsc.load_gather(bina, [idx16])
        take = key < cur
        plsc.store_scatter(bina, [idx16], jnp.where(take, key, cur))
        plsc.store_scatter(binx0, [idx16], x0_v[sl], mask=take)
        plsc.store_scatter(binx1, [idx16], x1_v[sl], mask=take)
        pwv = plsc.load_gather(pwfull, [obj])
        winv_v[sl] = jnp.where(noise, 0.0, 1.0 / (pwv + 1e-9))
        return 0
    lax.fori_loop(0, NV, p2, 0)
    pltpu.sync_copy(winv_v, winv_o.at[pl.ds(base, CH)])

    for kb in range(16):
        sl = pl.ds(kb * 16, 16)
        am = bina[sl]
        a0 = binx0[sl]
        a1 = binx1[sl]
        for c in range(1, 16):
            sc = pl.ds(c * K + kb * 16, 16)
            cm = bina[sc]
            upd = cm < am
            am = jnp.where(upd, cm, am)
            a0 = jnp.where(upd, binx0[sc], a0)
            a1 = jnp.where(upd, binx1[sc], a1)
        loci[sl] = am
        locx0[sl] = a0
        locx1[sl] = a1

    pltpu.sync_copy(loci, shm_amin.at[wid])
    pltpu.sync_copy(locx0, shm_x0.at[wid])
    pltpu.sync_copy(locx1, shm_x1.at[wid])
    plsc.subcore_barrier()

    @pl.when(wid == 0)
    def _():
        pltpu.sync_copy(shm_amin, combi)
        pltpu.sync_copy(shm_x0, combx0)
        pltpu.sync_copy(shm_x1, combx1)
        for kb in range(16):
            sl = pl.ds(kb * 16, 16)
            am = combi[0, sl]
            a0 = combx0[0, sl]
            a1 = combx1[0, sl]
            for r in range(1, NW):
                cm = combi[r, sl]
                upd = cm < am
                am = jnp.where(upd, cm, am)
                a0 = jnp.where(upd, combx0[r, sl], a0)
                a1 = jnp.where(upd, combx1[r, sl], a1)
            loci[sl] = am
            locx0[sl] = a0
            locx1[sl] = a1
        pltpu.sync_copy(loci, aidx_o)
        pltpu.sync_copy(locx0, xa0_o)
        pltpu.sync_copy(locx1, xa1_o)
        pltpu.sync_copy(segfull, segmax_o)
        pltpu.sync_copy(pwfull, pwsum_o)
        pltpu.sync_copy(shm_nc, combn)
        ncv = combn[0, :]
        for r in range(1, NW):
            ncv = ncv + combn[r, :]
        nc_v[...] = ncv
        pltpu.sync_copy(shm_nb, combn)
        nbv = combn[0, :]
        for r in range(1, NW):
            nbv = nbv + combn[r, :]
        nb_v[...] = nbv
        pltpu.sync_copy(nc_v, ncnt_o)
        pltpu.sync_copy(nb_v, nbeta_o)


_sc_call = functools.partial(
    pl.kernel,
    out_type=[
        jax.ShapeDtypeStruct((K,), jnp.float32),
        jax.ShapeDtypeStruct((K,), jnp.float32),
        jax.ShapeDtypeStruct((K,), jnp.int32),
        jax.ShapeDtypeStruct((K,), jnp.float32),
        jax.ShapeDtypeStruct((K,), jnp.float32),
        jax.ShapeDtypeStruct((16,), jnp.float32),
        jax.ShapeDtypeStruct((16,), jnp.float32),
        jax.ShapeDtypeStruct((N_PAD,), jnp.float32),
    ],
    mesh=plsc.VectorSubcoreMesh(core_axis_name="c", subcore_axis_name="s",
                                num_cores=1, num_subcores=16),
    compiler_params=pltpu.CompilerParams(needs_layout_passes=False),
    scratch_types=[
        pltpu.VMEM((CH,), jnp.int32),
        pltpu.VMEM((CH,), jnp.float32),
        pltpu.VMEM((CH,), jnp.float32),
        pltpu.VMEM((CH,), jnp.float32),
        pltpu.VMEM((CH,), jnp.float32),
        pltpu.VMEM((16 * K,), jnp.float32),
        pltpu.VMEM((16 * K,), jnp.float32),
        pltpu.VMEM((16 * K,), jnp.int32),
        pltpu.VMEM((16 * K,), jnp.float32),
        pltpu.VMEM((16 * K,), jnp.float32),
        pltpu.VMEM((K,), jnp.float32),
        pltpu.VMEM((K,), jnp.float32),
        pltpu.VMEM((K,), jnp.float32),
        pltpu.VMEM((K,), jnp.int32),
        pltpu.VMEM((K,), jnp.float32),
        pltpu.VMEM((K,), jnp.float32),
        pltpu.VMEM((NW, K), jnp.float32),
        pltpu.VMEM((NW, K), jnp.int32),
        pltpu.VMEM((NW, K), jnp.float32),
        pltpu.VMEM((NW, K), jnp.float32),
        pltpu.VMEM((16,), jnp.float32),
        pltpu.VMEM((16,), jnp.float32),
        pltpu.VMEM((NW, 16), jnp.float32),
        pltpu.VMEM_SHARED((NW, K), jnp.float32),
        pltpu.VMEM_SHARED((NW, K), jnp.float32),
        pltpu.VMEM_SHARED((NW, K), jnp.int32),
        pltpu.VMEM_SHARED((NW, K), jnp.float32),
        pltpu.VMEM_SHARED((NW, K), jnp.float32),
        pltpu.VMEM_SHARED((NW, 16), jnp.float32),
        pltpu.VMEM_SHARED((NW, 16), jnp.float32),
    ],
)(_sc_body)


def _tc_body(tidx_ref, beta_ref, x0_ref, x1_ref, winv_ref, feats_ref,
             segmax_ref, aidx_ref, xa0_ref, xa1_ref,
             ncnt_ref, nbeta_ref,
             out_ref,
             x0a_s, x1a_s, qa_s, acc_s):
    pid = pl.program_id(0)
    lanes = jax.lax.broadcasted_iota(jnp.int32, (1, BLK), 1)
    lane128 = jax.lax.broadcasted_iota(jnp.int32, (1, 128), 1)

    @pl.when(pid == 0)
    def _():
        validk = aidx_ref[...] < N
        valid = validk.astype(jnp.float32)
        beta_a = segmax_ref[...]
        atanh_a = 0.5 * jnp.log((1.0 + beta_a) / (1.0 - beta_a))
        qa_s[...] = jnp.where(validk, atanh_a * atanh_a + Q_MIN, 0.0)
        n_obj = jnp.maximum(jnp.sum(valid), 1.0)
        x0a_s[...] = xa0_ref[...]
        x1a_s[...] = xa1_ref[...]
        l_beta = jnp.sum(jnp.where(validk, 1.0 - beta_a, 0.0)) / n_obj
        n_noise = jnp.maximum(jnp.sum(ncnt_ref[...]), 1.0)
        l_noise = S_B * jnp.sum(nbeta_ref[...]) / n_noise
        acc_s[...] = (jnp.where(lane128 == 3, l_beta + l_noise, 0.0)
                      + jnp.where(lane128 == 5, n_obj, 0.0))

    tidx = tidx_ref[0]
    beta = jnp.clip(beta_ref[0], 1e-6, 1.0 - 1e-4)
    cols = jax.lax.broadcasted_iota(jnp.int32, (K, 1), 0)
    m = tidx == cols

    inb = lanes + pid * BLK < N
    q0 = 0.25 * jnp.log((1.0 + beta) / (1.0 - beta)) ** 2 + Q_MIN
    q = jnp.where(inb, q0, 0.0)
    f = feats_ref[0]
    en = f[0:1, :]
    ten = f[4:5, :]
    e_l = ((en - ten) / (jnp.abs(ten) + E_DEN_OFF)) ** 2
    dp0 = f[1:2, :] - f[5:6, :]
    dp1 = f[2:3, :] - f[6:7, :]
    pos_d = jnp.sqrt(dp0 * dp0 + dp1 * dp1 + 1e-6)
    pos_l = jnp.where(pos_d < 10.0, pos_d * pos_d,
                      100.0 + 20.0 * (pos_d - 10.0))
    dt = f[3:4, :] - f[7:8, :]
    w = beta * beta * (e_l + pos_l + dt * dt)

    x0r = x0_ref[0]
    x1r = x1_ref[0]
    xa0 = x0a_s[...]
    xa1 = x1a_s[...]
    xvec = jnp.concatenate([x0r, x1r], axis=0)
    xa_mat = jnp.concatenate([xa0, xa1], axis=1)
    cross = jax.lax.dot_general(
        xa_mat, xvec, (((1,), (0,)), ((), ())),
        preferred_element_type=jnp.float32)
    xa2 = xa0 * xa0 + xa1 * xa1
    x2 = x0r * x0r + x1r * x1r
    d2 = jnp.maximum(x2 + (xa2 - 2.0 * cross), 0.0)
    dist = jnp.sqrt(d2 + 1e-9)
    qq = q * qa_s[...]
    pot = qq * jnp.where(m, d2, jnp.maximum(1.0 - dist, 0.0))
    pay = w * winv_ref[0]
    acc_s[...] += (jnp.where(lane128 == 0, jnp.sum(pot), 0.0)
                   + jnp.where(lane128 == 4, jnp.sum(pay), 0.0))

    @pl.when(pid == NB - 1)
    def _():
        out_ref[...] = jnp.zeros((1, 128), jnp.float32) + jnp.where(
            lane128 == 0,
            acc_s[0, 0] / jnp.float32(N) + acc_s[0, 3]
            + acc_s[0, 4] / acc_s[0, 5], 0.0)


_ROW = pl.BlockSpec((1, 1, BLK), lambda i: (i, 0, 0))
_COLK = pl.BlockSpec((K, 1), lambda i: (0, 0))
_ROW16 = pl.BlockSpec((1, 16), lambda i: (0, 0))


def _prep(col, fill):
    return jnp.pad(col, (0, N_PAD - N), constant_values=fill).reshape(
        NB, 1, BLK)


@jax.jit
def _run(pred_beta, pred_ccoords, pred_energy, pred_pos, pred_time,
         t_idx, t_energy, t_pos, t_time):
    tidx_f = jnp.pad(t_idx[:, 0], (0, N_PAD - N), constant_values=-1)
    beta_f = jnp.pad(pred_beta[:, 0], (0, N_PAD - N))
    x0_f = jnp.pad(pred_ccoords[:, 0], (0, N_PAD - N))
    x1_f = jnp.pad(pred_ccoords[:, 1], (0, N_PAD - N))

    segmax, pwsum, aidx, xa0, xa1, ncnt, nbeta, winv = _sc_call(
        tidx_f, beta_f, x0_f, x1_f)

    feats = jnp.stack([
        pred_energy[:, 0], pred_pos[:, 0], pred_pos[:, 1], pred_time[:, 0],
        t_energy[:, 0], t_pos[:, 0], t_pos[:, 1], t_time[:, 0]])
    feats = jnp.pad(feats, ((0, 0), (0, N_PAD - N)))
    feats = feats.reshape(8, NB, BLK).transpose(1, 0, 2)

    ins = (
        tidx_f.reshape(NB, 1, BLK),
        beta_f.reshape(NB, 1, BLK),
        x0_f.reshape(NB, 1, BLK),
        x1_f.reshape(NB, 1, BLK),
        winv.reshape(NB, 1, BLK),
        feats,
        segmax.reshape(K, 1),
        aidx.reshape(K, 1),
        xa0.reshape(K, 1),
        xa1.reshape(K, 1),
        ncnt.reshape(1, 16),
        nbeta.reshape(1, 16),
    )
    out = pl.pallas_call(
        _tc_body,
        grid=(NB,),
        in_specs=[_ROW] * 5 + [pl.BlockSpec((1, 8, BLK), lambda i: (i, 0, 0))] + [_COLK] * 4 + [_ROW16] * 2,
        out_specs=pl.BlockSpec((1, 128), lambda i: (0, 0)),
        out_shape=jax.ShapeDtypeStruct((1, 128), jnp.float32),
        scratch_shapes=[
            pltpu.VMEM((K, 1), jnp.float32),
            pltpu.VMEM((K, 1), jnp.float32),
            pltpu.VMEM((K, 1), jnp.float32),
            pltpu.VMEM((1, 128), jnp.float32),
        ],
    )(*ins)
    return out[0, 0:1]


def kernel(pred_beta, pred_ccoords, pred_energy, pred_pos, pred_time,
           rechit_energy, t_idx, t_energy, t_pos, t_time, row_splits):
    lossval = _run(pred_beta, pred_ccoords, pred_energy, pred_pos, pred_time,
                   t_idx, t_energy, t_pos, t_time)
    return (pred_beta, lossval)


# speedup vs baseline: 1.0151x; 1.0151x over previous
import functools

import jax
import jax.numpy as jnp
from jax import lax
from jax.experimental import pallas as pl
from jax.experimental.pallas import tpu as pltpu
from jax.experimental.pallas import tpu_sc as plsc

N = 50000
K = 256
Q_MIN = 0.1
S_B = 1.0
E_DEN_OFF = 1.0
BLK = 12800
NB = 4
N_PAD = NB * BLK
BIG = 2 ** 30
NW = 16
CH = N_PAD // NW
NV = CH // 16


def _sc_body(tidx_hbm, beta_hbm, x0_hbm, x1_hbm,
             segmax_o, pwsum_o, aidx_o, xa0_o, xa1_o, ncnt_o, nbeta_o,
             winv_o,
             tidx_v, beta_v, x0_v, x1_v, winv_v,
             binm, binp, bina, binx0, binx1,
             segfull, pwfull, locf, loci, locx0, locx1,
             comb, combi, combx0, combx1, nc_v, nb_v, combn,
             shm_max, shm_pw, shm_amin, shm_x0, shm_x1, shm_nc, shm_nb):
    wid = lax.axis_index("s")
    base = wid * CH
    lane16 = lax.broadcasted_iota(jnp.int32, (16,), 0)

    pltpu.sync_copy(tidx_hbm.at[pl.ds(base, CH)], tidx_v)
    pltpu.sync_copy(beta_hbm.at[pl.ds(base, CH)], beta_v)
    pltpu.sync_copy(x0_hbm.at[pl.ds(base, CH)], x0_v)
    pltpu.sync_copy(x1_hbm.at[pl.ds(base, CH)], x1_v)

    def init_bins(i, _):
        sl = pl.ds(i * 16, 16)
        binm[sl] = jnp.full((16,), -1.0, jnp.float32)
        binp[sl] = jnp.zeros((16,), jnp.float32)
        bina[sl] = jnp.full((16,), BIG, jnp.int32)
        binx0[sl] = jnp.zeros((16,), jnp.float32)
        binx1[sl] = jnp.zeros((16,), jnp.float32)
        return 0
    lax.fori_loop(0, K, init_bins, 0)

    def p1(j, carry):
        nc, nb = carry
        sl = pl.ds(j * 16, 16)
        tv = tidx_v[sl]
        bv = jnp.minimum(jnp.maximum(beta_v[sl], 1e-6), 1.0 - 1e-4)
        noise = tv < 0
        obj = jnp.where(noise, 0, tv)
        idx16 = lane16 * K + obj
        val = jnp.where(noise, -1.0, bv)
        cur = plsc.load_gather(binm, [idx16])
        plsc.store_scatter(binm, [idx16], jnp.maximum(cur, val))
        pw = jnp.where(noise, 0.0, bv * bv)
        curp = plsc.load_gather(binp, [idx16])
        plsc.store_scatter(binp, [idx16], curp + pw)
        gi = base + j * 16 + lane16
        nf = jnp.where(jnp.logical_and(noise, gi < N), 1.0, 0.0)
        return (nc + nf, nb + nf * bv)

    nc, nb = lax.fori_loop(0, NV, p1, (jnp.zeros((16,), jnp.float32),
                                       jnp.zeros((16,), jnp.float32)))

    for kb in range(16):
        sl = pl.ds(kb * 16, 16)
        am = binm[sl]
        ap = binp[sl]
        for c in range(1, 16):
            sc = pl.ds(c * K + kb * 16, 16)
            am = jnp.maximum(am, binm[sc])
            ap = ap + binp[sc]
        segfull[sl] = am
        binp[sl] = ap

    nc_v[...] = nc
    nb_v[...] = nb
    pltpu.sync_copy(segfull, shm_max.at[wid])
    pltpu.sync_copy(binp.at[pl.ds(0, K)], shm_pw.at[wid])
    pltpu.sync_copy(nc_v, shm_nc.at[wid])
    pltpu.sync_copy(nb_v, shm_nb.at[wid])
    plsc.subcore_barrier()

    pltpu.sync_copy(shm_max, comb)
    for kb in range(16):
        sl = pl.ds(kb * 16, 16)
        am = comb[0, sl]
        for r in range(1, NW):
            am = jnp.maximum(am, comb[r, sl])
        segfull[sl] = am
    pltpu.sync_copy(shm_pw, comb)
    for kb in range(16):
        sl = pl.ds(kb * 16, 16)
        ap = comb[0, sl]
        for r in range(1, NW):
            ap = ap + comb[r, sl]
        pwfull[sl] = ap

    def p2(j, _):
        sl = pl.ds(j * 16, 16)
        tv = tidx_v[sl]
        bv = jnp.minimum(jnp.maximum(beta_v[sl], 1e-6), 1.0 - 1e-4)
        noise = tv < 0
        obj = jnp.where(noise, 0, tv)
        idx16 = lane16 * K + obj
        smax = plsc.load_gather(segfull, [obj])
        gi = base + j * 16 + lane16
        isal = jnp.logical_and(jnp.logical_not(noise), bv >= smax)
        key = jnp.where(isal, gi, BIG)
        cur = plsc.load_gather(bina, [idx16])
        take = key < cur
        plsc.store_scatter(bina, [idx16], jnp.where(take, key, cur))
        plsc.store_scatter(binx0, [idx16], x0_v[sl], mask=take)
        plsc.store_scatter(binx1, [idx16], x1_v[sl], mask=take)
        pwv = plsc.load_gather(pwfull, [obj])
        winv_v[sl] = jnp.where(noise, 0.0, 1.0 / (pwv + 1e-9))
        return 0
    lax.fori_loop(0, NV, p2, 0)
    pltpu.sync_copy(winv_v, winv_o.at[pl.ds(base, CH)])

    for kb in range(16):
        sl = pl.ds(kb * 16, 16)
        am = bina[sl]
        a0 = binx0[sl]
        a1 = binx1[sl]
        for c in range(1, 16):
            sc = pl.ds(c * K + kb * 16, 16)
            cm = bina[sc]
            upd = cm < am
            am = jnp.where(upd, cm, am)
            a0 = jnp.where(upd, binx0[sc], a0)
            a1 = jnp.where(upd, binx1[sc], a1)
        loci[sl] = am
        locx0[sl] = a0
        locx1[sl] = a1

    pltpu.sync_copy(loci, shm_amin.at[wid])
    pltpu.sync_copy(locx0, shm_x0.at[wid])
    pltpu.sync_copy(locx1, shm_x1.at[wid])
    plsc.subcore_barrier()

    @pl.when(wid == 0)
    def _():
        pltpu.sync_copy(shm_amin, combi)
        pltpu.sync_copy(shm_x0, combx0)
        pltpu.sync_copy(shm_x1, combx1)
        for kb in range(16):
            sl = pl.ds(kb * 16, 16)
            am = combi[0, sl]
            a0 = combx0[0, sl]
            a1 = combx1[0, sl]
            for r in range(1, NW):
                cm = combi[r, sl]
                upd = cm < am
                am = jnp.where(upd, cm, am)
                a0 = jnp.where(upd, combx0[r, sl], a0)
                a1 = jnp.where(upd, combx1[r, sl], a1)
            loci[sl] = am
            locx0[sl] = a0
            locx1[sl] = a1
        pltpu.sync_copy(loci, aidx_o)
        pltpu.sync_copy(locx0, xa0_o)
        pltpu.sync_copy(locx1, xa1_o)
        pltpu.sync_copy(segfull, segmax_o)
        pltpu.sync_copy(pwfull, pwsum_o)
        pltpu.sync_copy(shm_nc, combn)
        ncv = combn[0, :]
        for r in range(1, NW):
            ncv = ncv + combn[r, :]
        nc_v[...] = ncv
        pltpu.sync_copy(shm_nb, combn)
        nbv = combn[0, :]
        for r in range(1, NW):
            nbv = nbv + combn[r, :]
        nb_v[...] = nbv
        pltpu.sync_copy(nc_v, ncnt_o)
        pltpu.sync_copy(nb_v, nbeta_o)


_sc_call = functools.partial(
    pl.kernel,
    out_type=[
        jax.ShapeDtypeStruct((K,), jnp.float32),
        jax.ShapeDtypeStruct((K,), jnp.float32),
        jax.ShapeDtypeStruct((K,), jnp.int32),
        jax.ShapeDtypeStruct((K,), jnp.float32),
        jax.ShapeDtypeStruct((K,), jnp.float32),
        jax.ShapeDtypeStruct((16,), jnp.float32),
        jax.ShapeDtypeStruct((16,), jnp.float32),
        jax.ShapeDtypeStruct((N_PAD,), jnp.float32),
    ],
    mesh=plsc.VectorSubcoreMesh(core_axis_name="c", subcore_axis_name="s",
                                num_cores=1, num_subcores=16),
    compiler_params=pltpu.CompilerParams(needs_layout_passes=False),
    scratch_types=[
        pltpu.VMEM((CH,), jnp.int32),
        pltpu.VMEM((CH,), jnp.float32),
        pltpu.VMEM((CH,), jnp.float32),
        pltpu.VMEM((CH,), jnp.float32),
        pltpu.VMEM((CH,), jnp.float32),
        pltpu.VMEM((16 * K,), jnp.float32),
        pltpu.VMEM((16 * K,), jnp.float32),
        pltpu.VMEM((16 * K,), jnp.int32),
        pltpu.VMEM((16 * K,), jnp.float32),
        pltpu.VMEM((16 * K,), jnp.float32),
        pltpu.VMEM((K,), jnp.float32),
        pltpu.VMEM((K,), jnp.float32),
        pltpu.VMEM((K,), jnp.float32),
        pltpu.VMEM((K,), jnp.int32),
        pltpu.VMEM((K,), jnp.float32),
        pltpu.VMEM((K,), jnp.float32),
        pltpu.VMEM((NW, K), jnp.float32),
        pltpu.VMEM((NW, K), jnp.int32),
        pltpu.VMEM((NW, K), jnp.float32),
        pltpu.VMEM((NW, K), jnp.float32),
        pltpu.VMEM((16,), jnp.float32),
        pltpu.VMEM((16,), jnp.float32),
        pltpu.VMEM((NW, 16), jnp.float32),
        pltpu.VMEM_SHARED((NW, K), jnp.float32),
        pltpu.VMEM_SHARED((NW, K), jnp.float32),
        pltpu.VMEM_SHARED((NW, K), jnp.int32),
        pltpu.VMEM_SHARED((NW, K), jnp.float32),
        pltpu.VMEM_SHARED((NW, K), jnp.float32),
        pltpu.VMEM_SHARED((NW, 16), jnp.float32),
        pltpu.VMEM_SHARED((NW, 16), jnp.float32),
    ],
)(_sc_body)


def _tc_body(tidx_ref, beta_ref, x0_ref, x1_ref, winv_ref,
             en_ref, p0_ref, p1_ref,
             tim_ref, ten_ref, tp0_ref, tp1_ref, ttim_ref,
             segmax_ref, aidx_ref, xa0_ref, xa1_ref,
             ncnt_ref, nbeta_ref,
             out_ref,
             x0a_s, x1a_s, qa_s, acc_s):
    pid = pl.program_id(0)
    lanes = jax.lax.broadcasted_iota(jnp.int32, (1, BLK), 1)
    lane128 = jax.lax.broadcasted_iota(jnp.int32, (1, 128), 1)

    @pl.when(pid == 0)
    def _():
        validk = aidx_ref[...] < N
        valid = validk.astype(jnp.float32)
        beta_a = segmax_ref[...]
        atanh_a = 0.5 * jnp.log((1.0 + beta_a) / (1.0 - beta_a))
        qa_s[...] = jnp.where(validk, atanh_a * atanh_a + Q_MIN, 0.0)
        n_obj = jnp.maximum(jnp.sum(valid), 1.0)
        x0a_s[...] = xa0_ref[...]
        x1a_s[...] = xa1_ref[...]
        l_beta = jnp.sum(jnp.where(validk, 1.0 - beta_a, 0.0)) / n_obj
        n_noise = jnp.maximum(jnp.sum(ncnt_ref[...]), 1.0)
        l_noise = S_B * jnp.sum(nbeta_ref[...]) / n_noise
        acc_s[...] = (jnp.where(lane128 == 3, l_beta + l_noise, 0.0)
                      + jnp.where(lane128 == 5, n_obj, 0.0))

    tidx = tidx_ref[0]
    beta = jnp.clip(beta_ref[0], 1e-6, 1.0 - 1e-4)
    cols = jax.lax.broadcasted_iota(jnp.int32, (K, 1), 0)
    m = tidx == cols

    inb = lanes + pid * BLK < N
    q0 = 0.25 * jnp.log((1.0 + beta) / (1.0 - beta)) ** 2 + Q_MIN
    q = jnp.where(inb, q0, 0.0)
    en = en_ref[0]
    ten = ten_ref[0]
    e_l = ((en - ten) / (jnp.abs(ten) + E_DEN_OFF)) ** 2
    dp0 = p0_ref[0] - tp0_ref[0]
    dp1 = p1_ref[0] - tp1_ref[0]
    pos_d = jnp.sqrt(dp0 * dp0 + dp1 * dp1 + 1e-6)
    pos_l = jnp.where(pos_d < 10.0, pos_d * pos_d,
                      100.0 + 20.0 * (pos_d - 10.0))
    dt = tim_ref[0] - ttim_ref[0]
    w = beta * beta * (e_l + pos_l + dt * dt)

    x0r = x0_ref[0]
    x1r = x1_ref[0]
    xa0 = x0a_s[...]
    xa1 = x1a_s[...]
    xvec = jnp.concatenate([x0r, x1r], axis=0)
    xa_mat = jnp.concatenate([xa0, xa1], axis=1)
    cross = jax.lax.dot_general(
        xa_mat, xvec, (((1,), (0,)), ((), ())),
        preferred_element_type=jnp.float32)
    xa2 = xa0 * xa0 + xa1 * xa1
    x2 = x0r * x0r + x1r * x1r
    d2 = jnp.maximum(x2 + (xa2 - 2.0 * cross), 0.0)
    dist = jnp.sqrt(d2 + 1e-9)
    qq = q * qa_s[...]
    pot = qq * jnp.where(m, d2, jnp.maximum(1.0 - dist, 0.0))
    pay = w * winv_ref[0]
    acc_s[...] += (jnp.where(lane128 == 0, jnp.sum(pot), 0.0)
                   + jnp.where(lane128 == 4, jnp.sum(pay), 0.0))

    @pl.when(pid == NB - 1)
    def _():
        out_ref[...] = jnp.zeros((1, 128), jnp.float32) + jnp.where(
            lane128 == 0,
            acc_s[0, 0] / jnp.float32(N) + acc_s[0, 3]
            + acc_s[0, 4] / acc_s[0, 5], 0.0)


_ROW = pl.BlockSpec((1, 1, BLK), lambda i: (i, 0, 0))
_COLK = pl.BlockSpec((K, 1), lambda i: (0, 0))
_ROW16 = pl.BlockSpec((1, 16), lambda i: (0, 0))


def _prep(col, fill):
    return jnp.pad(col, (0, N_PAD - N), constant_values=fill).reshape(
        NB, 1, BLK)


@jax.jit
def _run(pred_beta, pred_ccoords, pred_energy, pred_pos, pred_time,
         t_idx, t_energy, t_pos, t_time):
    tidx_f = jnp.pad(t_idx[:, 0], (0, N_PAD - N), constant_values=-1)
    beta_f = jnp.pad(pred_beta[:, 0], (0, N_PAD - N))
    x0_f = jnp.pad(pred_ccoords[:, 0], (0, N_PAD - N))
    x1_f = jnp.pad(pred_ccoords[:, 1], (0, N_PAD - N))

    segmax, pwsum, aidx, xa0, xa1, ncnt, nbeta, winv = _sc_call(
        tidx_f, beta_f, x0_f, x1_f)

    ins = (
        tidx_f.reshape(NB, 1, BLK),
        beta_f.reshape(NB, 1, BLK),
        x0_f.reshape(NB, 1, BLK),
        x1_f.reshape(NB, 1, BLK),
        winv.reshape(NB, 1, BLK),
        _prep(pred_energy[:, 0], 0.0),
        _prep(pred_pos[:, 0], 0.0),
        _prep(pred_pos[:, 1], 0.0),
        _prep(pred_time[:, 0], 0.0),
        _prep(t_energy[:, 0], 0.0),
        _prep(t_pos[:, 0], 0.0),
        _prep(t_pos[:, 1], 0.0),
        _prep(t_time[:, 0], 0.0),
        segmax.reshape(K, 1),
        aidx.reshape(K, 1),
        xa0.reshape(K, 1),
        xa1.reshape(K, 1),
        ncnt.reshape(1, 16),
        nbeta.reshape(1, 16),
    )
    out = pl.pallas_call(
        _tc_body,
        grid=(NB,),
        in_specs=[_ROW] * 13 + [_COLK] * 4 + [_ROW16] * 2,
        out_specs=pl.BlockSpec((1, 128), lambda i: (0, 0)),
        out_shape=jax.ShapeDtypeStruct((1, 128), jnp.float32),
        scratch_shapes=[
            pltpu.VMEM((K, 1), jnp.float32),
            pltpu.VMEM((K, 1), jnp.float32),
            pltpu.VMEM((K, 1), jnp.float32),
            pltpu.VMEM((1, 128), jnp.float32),
        ],
    )(*ins)
    return out[0, 0:1]


def kernel(pred_beta, pred_ccoords, pred_energy, pred_pos, pred_time,
           rechit_energy, t_idx, t_energy, t_pos, t_time, row_splits):
    lossval = _run(pred_beta, pred_ccoords, pred_energy, pred_pos, pred_time,
                   t_idx, t_energy, t_pos, t_time)
    return (pred_beta, lossval)
